# baseline (speedup 1.0000x reference)
import functools

import jax
import jax.numpy as jnp
from jax import lax
from jax.experimental import pallas as pl
from jax.experimental.pallas import tpu as pltpu
from jax.experimental.pallas import tpu_sc as plsc

N = 10000
E = 640000
D_IN = 128
H = 128
G = 64

NC = 2
NS = 16
NW = NC * NS

NPAD = 10112
PADROW = 10008
CH = 128
CPT = 160
EPAD = NW * CPT * CH
NBUF = 4
PRE = 2

RB = 2528

_mesh = plsc.VectorSubcoreMesh(core_axis_name="c", subcore_axis_name="s")
_RPT = NPAD // NS


def _sc_conv(zs, srcp, dstp, zeros64):
    @functools.partial(
        pl.kernel,
        out_type=jax.ShapeDtypeStruct((NC, NPAD, 64), jnp.float32),
        mesh=_mesh,
        compiler_params=pltpu.CompilerParams(use_tc_tiling_on_sc=False),
        scratch_types=[
            pltpu.VMEM((CPT, CH), jnp.int32),
            pltpu.VMEM((CPT, CH), jnp.int32),
            pltpu.VMEM((NBUF, CH, 64), jnp.float32),
            pltpu.VMEM_SHARED((NPAD, 64), jnp.float32),
            pltpu.SemaphoreType.DMA,
            pltpu.SemaphoreType.DMA,
        ],
    )
    def k(zs_hbm, srcp_hbm, dstp_hbm, z64_hbm, out_hbm,
          src_v, dst_v, rows_v, acc_sh, gsem, ssem):
        c = lax.axis_index("c")
        s = lax.axis_index("s")
        w = c * NS + s
        pltpu.sync_copy(z64_hbm.at[pl.ds(s * _RPT, _RPT)],
                        acc_sh.at[pl.ds(s * _RPT, _RPT)])
        pltpu.sync_copy(srcp_hbm.at[w], src_v)
        pltpu.sync_copy(dstp_hbm.at[w], dst_v)
        plsc.subcore_barrier()

        for b in range(PRE):
            pltpu.async_copy(zs_hbm.at[src_v.at[b]], rows_v.at[b], gsem)

        @pl.loop(0, CPT, step=NBUF)
        def _(j):
            for b in range(NBUF):
                jj = j + b

                @pl.when(jj >= NBUF - PRE)
                def _():
                    pltpu.make_async_copy(
                        rows_v.at[0], acc_sh.at[dst_v.at[0]], ssem).wait()

                nxt = jj + PRE

                @pl.when(nxt < CPT)
                def _():
                    pltpu.async_copy(zs_hbm.at[src_v.at[nxt]],
                                     rows_v.at[(b + PRE) % NBUF], gsem)

                pltpu.make_async_copy(
                    zs_hbm.at[src_v.at[jj]], rows_v.at[b], gsem).wait()
                pltpu.async_copy(rows_v.at[b], acc_sh.at[dst_v.at[jj]],
                                 ssem, add=True)

        for _ in range(NBUF - PRE):
            pltpu.make_async_copy(
                rows_v.at[0], acc_sh.at[dst_v.at[0]], ssem).wait()

        plsc.subcore_barrier()
        pltpu.sync_copy(acc_sh.at[pl.ds(s * _RPT, _RPT)],
                        out_hbm.at[c, pl.ds(s * _RPT, _RPT)])

    return k(zs, srcp, dstp, zeros64)


def _sc_degree(dstp, ones16, zeros16):
    @functools.partial(
        pl.kernel,
        out_type=jax.ShapeDtypeStruct((NC, NPAD, 16), jnp.float32),
        mesh=_mesh,
        compiler_params=pltpu.CompilerParams(use_tc_tiling_on_sc=False),
        scratch_types=[
            pltpu.VMEM((CPT, CH), jnp.int32),
            pltpu.VMEM((CH, 16), jnp.float32),
            pltpu.VMEM_SHARED((NPAD, 16), jnp.float32),
        ],
    )
    def k(dstp_hbm, ones_hbm, z16_hbm, out_hbm, dst_v, ones_v, deg_sh):
        c = lax.axis_index("c")
        s = lax.axis_index("s")
        w = c * NS + s
        pltpu.sync_copy(z16_hbm.at[pl.ds(s * _RPT, _RPT)],
                        deg_sh.at[pl.ds(s * _RPT, _RPT)])
        pltpu.sync_copy(dstp_hbm.at[w], dst_v)
        pltpu.sync_copy(ones_hbm, ones_v)
        plsc.subcore_barrier()

        @pl.loop(0, CPT)
        def _(j):
            pltpu.sync_copy(ones_v, deg_sh.at[dst_v.at[j]], add=True)

        plsc.subcore_barrier()
        pltpu.sync_copy(deg_sh.at[pl.ds(s * _RPT, _RPT)],
                        out_hbm.at[c, pl.ds(s * _RPT, _RPT)])

    return k(dstp, ones16, zeros16)


def _tc_lin1(xp, W1, b1, Wc1):
    def body(x_ref, w1_ref, b1_ref, wc1_ref, z1_ref):
        h = jnp.dot(x_ref[...], w1_ref[...],
                    preferred_element_type=jnp.float32) + b1_ref[...]
        h = jnp.maximum(h, 0.0)
        z1_ref[...] = jnp.dot(h, wc1_ref[...],
                              preferred_element_type=jnp.float32)

    return pl.pallas_call(
        body,
        grid=(NPAD // RB,),
        in_specs=[
            pl.BlockSpec((RB, D_IN), lambda i: (i, 0)),
            pl.BlockSpec((D_IN, H), lambda i: (0, 0)),
            pl.BlockSpec((1, H), lambda i: (0, 0)),
            pl.BlockSpec((H, H // 2), lambda i: (0, 0)),
        ],
        out_specs=pl.BlockSpec((RB, H // 2), lambda i: (i, 0)),
        out_shape=jax.ShapeDtypeStruct((NPAD, H // 2), jnp.float32),
    )(xp, W1, b1.reshape(1, H), Wc1)


def _tc_norm(degp, z1):
    def body(deg_ref, z1_ref, dinv_ref, zs1_ref):
        d = deg_ref[...]
        deg = d[0] + d[1] + 1.0
        dinv = lax.rsqrt(deg)
        dinv_ref[...] = dinv
        zs1_ref[...] = z1_ref[...] * dinv[:, :1]

    return pl.pallas_call(
        body,
        grid=(NPAD // RB,),
        in_specs=[
            pl.BlockSpec((NC, RB, 16), lambda i: (0, i, 0)),
            pl.BlockSpec((RB, H // 2), lambda i: (i, 0)),
        ],
        out_specs=[
            pl.BlockSpec((RB, 16), lambda i: (i, 0)),
            pl.BlockSpec((RB, H // 2), lambda i: (i, 0)),
        ],
        out_shape=[
            jax.ShapeDtypeStruct((NPAD, 16), jnp.float32),
            jax.ShapeDtypeStruct((NPAD, H // 2), jnp.float32),
        ],
    )(degp, z1)


def _tc_mid(acc1, z1, dinv16, bc1, Wc2):
    def body(acc_ref, z1_ref, dinv_ref, bc1_ref, wc2_ref, z2_ref, zs2_ref):
        a = acc_ref[...]
        dv = dinv_ref[...][:, :1]
        h1 = jnp.maximum(dv * (a[0] + a[1]) + dv * dv * z1_ref[...]
                         + bc1_ref[...], 0.0)
        z2 = jnp.dot(h1, wc2_ref[...], preferred_element_type=jnp.float32)
        z2_ref[...] = z2
        zs2_ref[...] = z2 * dv

    return pl.pallas_call(
        body,
        grid=(NPAD // RB,),
        in_specs=[
            pl.BlockSpec((NC, RB, H // 2), lambda i: (0, i, 0)),
            pl.BlockSpec((RB, H // 2), lambda i: (i, 0)),
            pl.BlockSpec((RB, 16), lambda i: (i, 0)),
            pl.BlockSpec((1, H // 2), lambda i: (0, 0)),
            pl.BlockSpec((H // 2, H // 2), lambda i: (0, 0)),
        ],
        out_specs=[
            pl.BlockSpec((RB, H // 2), lambda i: (i, 0)),
            pl.BlockSpec((RB, H // 2), lambda i: (i, 0)),
        ],
        out_shape=[
            jax.ShapeDtypeStruct((NPAD, H // 2), jnp.float32),
            jax.ShapeDtypeStruct((NPAD, H // 2), jnp.float32),
        ],
    )(acc1, z1, dinv16, bc1.reshape(1, H // 2), Wc2)


def _tc_head(acc2, z2, dinv16, bc2, batch2d, W2, b2, W3, b3):
    def body(acc_ref, z2_ref, dinv_ref, bc2_ref, batch_ref,
             w2_ref, b2_ref, w3_ref, b3_ref, out_ref):
        a = acc_ref[...]
        dv = dinv_ref[...][:, :1]
        h2 = jnp.maximum(dv * (a[0] + a[1]) + dv * dv * z2_ref[...]
                         + bc2_ref[...], 0.0)
        gi = lax.broadcasted_iota(jnp.int32, (G, NPAD), 0)
        m = (batch_ref[...] == gi).astype(jnp.float32)
        sums = jnp.dot(m, h2, preferred_element_type=jnp.float32)
        cnts = jnp.sum(m, axis=1, keepdims=True)
        pooled = sums / jnp.maximum(cnts, 1.0)
        r = jnp.maximum(jnp.dot(pooled, w2_ref[...],
                                preferred_element_type=jnp.float32)
                        + b2_ref[...], 0.0)
        out_ref[...] = jnp.dot(r, w3_ref[...],
                               preferred_element_type=jnp.float32) + b3_ref[...]

    return pl.pallas_call(
        body,
        out_shape=jax.ShapeDtypeStruct((G, 1), jnp.float32),
    )(acc2, z2, dinv16, bc2.reshape(1, H // 2), batch2d,
      W2, b2.reshape(1, H // 4), W3, b3.reshape(1, 1))


def kernel(x, edge_index, batch, W1, b1, Wc1, bc1, Wc2, bc2, W2, b2, W3, b3):
    xp = jnp.pad(x, ((0, NPAD - N), (0, 0)))
    pad = EPAD - E
    srcp = jnp.concatenate(
        [edge_index[0], jnp.zeros((pad,), jnp.int32)]).reshape(NW, CPT, CH)
    dstp = jnp.concatenate(
        [edge_index[1], jnp.full((pad,), PADROW, jnp.int32)]).reshape(NW, CPT, CH)
    batch2d = jnp.pad(batch, (0, NPAD - N), constant_values=G).reshape(1, NPAD)
    zeros64 = jnp.zeros((NPAD, 64), jnp.float32)
    zeros16 = jnp.zeros((NPAD, 16), jnp.float32)
    ones16 = jnp.ones((CH, 16), jnp.float32)

    degp = _sc_degree(dstp, ones16, zeros16)
    z1 = _tc_lin1(xp, W1, b1, Wc1)
    dinv16, zs1 = _tc_norm(degp, z1)

    acc1 = _sc_conv(zs1, srcp, dstp, zeros64)
    z2, zs2 = _tc_mid(acc1, z1, dinv16, bc1, Wc2)

    acc2 = _sc_conv(zs2, srcp, dstp, zeros64)
    return _tc_head(acc2, z2, dinv16, bc2, batch2d, W2, b2, W3, b3)


# speedup vs baseline: 1.8964x; 1.8964x over previous
import functools

import jax
import jax.numpy as jnp
from jax import lax
from jax.experimental import pallas as pl
from jax.experimental.pallas import tpu as pltpu
from jax.experimental.pallas import tpu_sc as plsc

N = 10000
E = 640000
D_IN = 128
H = 128
G = 64

NC = 2
NS = 16
NW = NC * NS

NPAD = 10112
PADROW = 10008
CH = 128
NBUF = 4
PRE = 3

CPTC = 316
EPADC = NS * CPTC * CH
CPTD = 160
EPADD = NW * CPTD * CH

RB = 2528
F2 = 32

_mesh = plsc.VectorSubcoreMesh(core_axis_name="c", subcore_axis_name="s")
_RPT = NPAD // NS
_SC_PARAMS = pltpu.CompilerParams(use_tc_tiling_on_sc=False)


def _sc_conv(zsp, srcp, dstp, zeros32):
    @functools.partial(
        pl.kernel,
        out_type=jax.ShapeDtypeStruct((NC, NPAD, F2), jnp.float32),
        mesh=_mesh,
        compiler_params=_SC_PARAMS,
        scratch_types=[
            pltpu.VMEM((CPTC, CH), jnp.int32),
            pltpu.VMEM((CPTC, CH), jnp.int32),
            pltpu.VMEM((NBUF, CH, F2), jnp.float32),
            pltpu.VMEM_SHARED((NPAD, F2), jnp.float32),
            pltpu.VMEM_SHARED((NPAD, F2), jnp.float32),
            pltpu.SemaphoreType.DMA,
        ],
    )
    def k(zsp_hbm, srcp_hbm, dstp_hbm, z32_hbm, out_hbm,
          src_v, dst_v, rows_v, acc_sh, zs_sh, gsem):
        c = lax.axis_index("c")
        s = lax.axis_index("s")
        sl = pl.ds(s * _RPT, _RPT)
        pltpu.sync_copy(z32_hbm, acc_sh.at[sl])
        pltpu.sync_copy(zsp_hbm.at[c, sl], zs_sh.at[sl])
        pltpu.sync_copy(srcp_hbm.at[s], src_v)
        pltpu.sync_copy(dstp_hbm.at[s], dst_v)
        plsc.subcore_barrier()

        for b in range(PRE):
            pltpu.async_copy(zs_sh.at[src_v.at[b]], rows_v.at[b], gsem)

        @pl.loop(0, CPTC, step=NBUF)
        def _(j):
            for b in range(NBUF):
                jj = j + b
                pltpu.make_async_copy(
                    zs_sh.at[src_v.at[jj]], rows_v.at[b], gsem).wait()
                pltpu.sync_copy(rows_v.at[b], acc_sh.at[dst_v.at[jj]],
                                add=True)
                nxt = jj + PRE

                @pl.when(nxt < CPTC)
                def _():
                    pltpu.async_copy(zs_sh.at[src_v.at[nxt]],
                                     rows_v.at[(b + PRE) % NBUF], gsem)

        plsc.subcore_barrier()
        pltpu.sync_copy(acc_sh.at[sl], out_hbm.at[c, sl])

    return k(zsp, srcp, dstp, zeros32)


def _sc_conv_hbm(zflat, srcpo, dstp, zeros32):
    @functools.partial(
        pl.kernel,
        out_type=jax.ShapeDtypeStruct((NC, NPAD, F2), jnp.float32),
        mesh=_mesh,
        compiler_params=_SC_PARAMS,
        scratch_types=[
            pltpu.VMEM((CPTC, CH), jnp.int32),
            pltpu.VMEM((CPTC, CH), jnp.int32),
            pltpu.VMEM((NBUF, CH, F2), jnp.float32),
            pltpu.VMEM_SHARED((NPAD, F2), jnp.float32),
            pltpu.SemaphoreType.DMA,
        ],
    )
    def k(zf_hbm, srcp_hbm, dstp_hbm, z32_hbm, out_hbm,
          src_v, dst_v, rows_v, acc_sh, gsem):
        c = lax.axis_index("c")
        s = lax.axis_index("s")
        sl = pl.ds(s * _RPT, _RPT)
        pltpu.sync_copy(z32_hbm, acc_sh.at[sl])
        pltpu.sync_copy(srcp_hbm.at[c, s], src_v)
        pltpu.sync_copy(dstp_hbm.at[s], dst_v)
        plsc.subcore_barrier()

        for b in range(PRE):
            pltpu.async_copy(zf_hbm.at[src_v.at[b]], rows_v.at[b], gsem)

        @pl.loop(0, CPTC, step=NBUF)
        def _(j):
            for b in range(NBUF):
                jj = j + b
                pltpu.make_async_copy(
                    zf_hbm.at[src_v.at[jj]], rows_v.at[b], gsem).wait()
                pltpu.sync_copy(rows_v.at[b], acc_sh.at[dst_v.at[jj]],
                                add=True)
                nxt = jj + PRE

                @pl.when(nxt < CPTC)
                def _():
                    pltpu.async_copy(zf_hbm.at[src_v.at[nxt]],
                                     rows_v.at[(b + PRE) % NBUF], gsem)

        plsc.subcore_barrier()
        pltpu.sync_copy(acc_sh.at[sl], out_hbm.at[c, sl])

    return k(zflat, srcpo, dstp, zeros32)


def _sc_degree(dstp, ones16, zeros16):
    @functools.partial(
        pl.kernel,
        out_type=jax.ShapeDtypeStruct((NC, NPAD, 16), jnp.float32),
        mesh=_mesh,
        compiler_params=_SC_PARAMS,
        scratch_types=[
            pltpu.VMEM((CPTD, CH), jnp.int32),
            pltpu.VMEM((CH, 16), jnp.float32),
            pltpu.VMEM_SHARED((NPAD, 16), jnp.float32),
        ],
    )
    def k(dstp_hbm, ones_hbm, z16_hbm, out_hbm, dst_v, ones_v, deg_sh):
        c = lax.axis_index("c")
        s = lax.axis_index("s")
        w = c * NS + s
        sl = pl.ds(s * _RPT, _RPT)
        pltpu.sync_copy(z16_hbm, deg_sh.at[sl])
        pltpu.sync_copy(dstp_hbm.at[w], dst_v)
        pltpu.sync_copy(ones_hbm, ones_v)
        plsc.subcore_barrier()

        @pl.loop(0, CPTD)
        def _(j):
            pltpu.sync_copy(ones_v, deg_sh.at[dst_v.at[j]], add=True)

        plsc.subcore_barrier()
        pltpu.sync_copy(deg_sh.at[sl], out_hbm.at[c, sl])

    return k(dstp, ones16, zeros16)


def _split2(z):
    return jnp.concatenate([z[None, :, :F2], z[None, :, F2:]], axis=0)


def _tc_lin1(xp, W1, b1, Wc1):
    def body(x_ref, w1_ref, b1_ref, wc1_ref, z1_ref):
        h = jnp.dot(x_ref[...], w1_ref[...],
                    preferred_element_type=jnp.float32) + b1_ref[...]
        h = jnp.maximum(h, 0.0)
        z1_ref[...] = jnp.dot(h, wc1_ref[...],
                              preferred_element_type=jnp.float32)

    return pl.pallas_call(
        body,
        grid=(NPAD // RB,),
        in_specs=[
            pl.BlockSpec((RB, D_IN), lambda i: (i, 0)),
            pl.BlockSpec((D_IN, H), lambda i: (0, 0)),
            pl.BlockSpec((1, H), lambda i: (0, 0)),
            pl.BlockSpec((H, H // 2), lambda i: (0, 0)),
        ],
        out_specs=pl.BlockSpec((RB, H // 2), lambda i: (i, 0)),
        out_shape=jax.ShapeDtypeStruct((NPAD, H // 2), jnp.float32),
    )(xp, W1, b1.reshape(1, H), Wc1)


def _tc_norm(degp, z1):
    def body(deg_ref, z1_ref, dinv_ref, zs1_ref):
        d = deg_ref[...]
        deg = d[0] + d[1] + 1.0
        dinv = lax.rsqrt(deg)
        dinv_ref[...] = dinv
        zs1_ref[...] = _split2(z1_ref[...] * dinv[:, :1])

    return pl.pallas_call(
        body,
        grid=(NPAD // RB,),
        in_specs=[
            pl.BlockSpec((NC, RB, 16), lambda i: (0, i, 0)),
            pl.BlockSpec((RB, H // 2), lambda i: (i, 0)),
        ],
        out_specs=[
            pl.BlockSpec((RB, 16), lambda i: (i, 0)),
            pl.BlockSpec((NC, RB, F2), lambda i: (0, i, 0)),
        ],
        out_shape=[
            jax.ShapeDtypeStruct((NPAD, 16), jnp.float32),
            jax.ShapeDtypeStruct((NC, NPAD, F2), jnp.float32),
        ],
    )(degp, z1)


def _tc_mid(acc1, z1, dinv16, bc1, Wc2):
    def body(acc_ref, z1_ref, dinv_ref, bc1_ref, wc2_ref, z2_ref, zs2_ref):
        a = acc_ref[...]
        acc = jnp.concatenate([a[0], a[1]], axis=1)
        dv = dinv_ref[...][:, :1]
        h1 = jnp.maximum(dv * acc + dv * dv * z1_ref[...]
                         + bc1_ref[...], 0.0)
        z2 = jnp.dot(h1, wc2_ref[...], preferred_element_type=jnp.float32)
        z2_ref[...] = z2
        zs2_ref[...] = _split2(z2 * dv)

    return pl.pallas_call(
        body,
        grid=(NPAD // RB,),
        in_specs=[
            pl.BlockSpec((NC, RB, F2), lambda i: (0, i, 0)),
            pl.BlockSpec((RB, H // 2), lambda i: (i, 0)),
            pl.BlockSpec((RB, 16), lambda i: (i, 0)),
            pl.BlockSpec((1, H // 2), lambda i: (0, 0)),
            pl.BlockSpec((H // 2, H // 2), lambda i: (0, 0)),
        ],
        out_specs=[
            pl.BlockSpec((RB, H // 2), lambda i: (i, 0)),
            pl.BlockSpec((NC, RB, F2), lambda i: (0, i, 0)),
        ],
        out_shape=[
            jax.ShapeDtypeStruct((NPAD, H // 2), jnp.float32),
            jax.ShapeDtypeStruct((NC, NPAD, F2), jnp.float32),
        ],
    )(acc1, z1, dinv16, bc1.reshape(1, H // 2), Wc2)


def _tc_head(acc2, z2, dinv16, bc2, batch2d, W2, b2, W3, b3):
    def body(acc_ref, z2_ref, dinv_ref, bc2_ref, batch_ref,
             w2_ref, b2_ref, w3_ref, b3_ref, out_ref):
        a = acc_ref[...]
        acc = jnp.concatenate([a[0], a[1]], axis=1)
        dv = dinv_ref[...][:, :1]
        h2 = jnp.maximum(dv * acc + dv * dv * z2_ref[...]
                         + bc2_ref[...], 0.0)
        gi = lax.broadcasted_iota(jnp.int32, (G, NPAD), 0)
        m = (batch_ref[...] == gi).astype(jnp.float32)
        sums = jnp.dot(m, h2, preferred_element_type=jnp.float32)
        cnts = jnp.sum(m, axis=1, keepdims=True)
        pooled = sums / jnp.maximum(cnts, 1.0)
        r = jnp.maximum(jnp.dot(pooled, w2_ref[...],
                                preferred_element_type=jnp.float32)
                        + b2_ref[...], 0.0)
        out_ref[...] = jnp.dot(r, w3_ref[...],
                               preferred_element_type=jnp.float32) + b3_ref[...]

    return pl.pallas_call(
        body,
        out_shape=jax.ShapeDtypeStruct((G, 1), jnp.float32),
    )(acc2, z2, dinv16, bc2.reshape(1, H // 2), batch2d,
      W2, b2.reshape(1, H // 4), W3, b3.reshape(1, 1))


def kernel(x, edge_index, batch, W1, b1, Wc1, bc1, Wc2, bc2, W2, b2, W3, b3):
    xp = jnp.pad(x, ((0, NPAD - N), (0, 0)))
    padc = EPADC - E
    srcp = jnp.concatenate(
        [edge_index[0], jnp.zeros((padc,), jnp.int32)]).reshape(NS, CPTC, CH)
    dstpc = jnp.concatenate(
        [edge_index[1], jnp.full((padc,), PADROW, jnp.int32)]).reshape(NS, CPTC, CH)
    padd = EPADD - E
    dstpd = jnp.concatenate(
        [edge_index[1], jnp.full((padd,), PADROW, jnp.int32)]).reshape(NW, CPTD, CH)
    batch2d = jnp.pad(batch, (0, NPAD - N), constant_values=G).reshape(1, NPAD)
    zeros32 = jnp.zeros((_RPT, F2), jnp.float32)
    zeros16 = jnp.zeros((_RPT, 16), jnp.float32)
    ones16 = jnp.ones((CH, 16), jnp.float32)

    degp = _sc_degree(dstpd, ones16, zeros16)
    z1 = _tc_lin1(xp, W1, b1, Wc1)
    dinv16, zs1 = _tc_norm(degp, z1)

    srcpo = srcp[None] + (jnp.arange(NC, dtype=jnp.int32) * NPAD
                          ).reshape(NC, 1, 1, 1)
    acc1 = _sc_conv_hbm(zs1.reshape(NC * NPAD, F2), srcpo, dstpc, zeros32)
    z2, zs2 = _tc_mid(acc1, z1, dinv16, bc1, Wc2)

    acc2 = _sc_conv_hbm(zs2.reshape(NC * NPAD, F2), srcpo, dstpc, zeros32)
    return _tc_head(acc2, z2, dinv16, bc2, batch2d, W2, b2, W3, b3)
